# baseline (speedup 1.0000x reference)
import functools

import jax
import jax.numpy as jnp
from jax import lax
from jax.experimental import pallas as pl
from jax.experimental.pallas import tpu as pltpu
from jax.experimental.pallas import tpu_sc as plsc

B_ROWS = 16384 * 50
DIM = 32
NUM_CORES = 2
NUM_SUBCORES = 16
NW = NUM_CORES * NUM_SUBCORES
ROWS_PER_W = B_ROWS // NW
CHUNK = 1024
NCHUNK = ROWS_PER_W // CHUNK

_mesh = plsc.VectorSubcoreMesh(core_axis_name="c", subcore_axis_name="s")


@functools.partial(
    pl.kernel,
    mesh=_mesh,
    compiler_params=pltpu.CompilerParams(use_tc_tiling_on_sc=False),
    out_type=jax.ShapeDtypeStruct((B_ROWS, DIM), jnp.float32),
    scratch_types=[
        pltpu.VMEM_SHARED((NUM_SUBCORES, ROWS_PER_W), jnp.int32),
        pltpu.SMEM((CHUNK,), jnp.int32),
        pltpu.SemaphoreType.DMA,
    ],
)
def _emb_lookup(idx_hbm, w_hbm, out_hbm, idx_sh, idx_s, sem):
    cid = lax.axis_index("c")
    sid = lax.axis_index("s")
    wid = sid * NUM_CORES + cid
    base = wid * ROWS_PER_W

    pltpu.sync_copy(idx_hbm.at[pl.ds(base, ROWS_PER_W)], idx_sh.at[sid])

    def chunk_body(c, carry):
        off = base + c * CHUNK
        pltpu.sync_copy(idx_sh.at[sid].at[pl.ds(c * CHUNK, CHUNK)], idx_s)

        def row_body(j, carry2):
            pltpu.async_copy(w_hbm.at[pl.ds(idx_s[j], 1)],
                             out_hbm.at[pl.ds(off + j, 1)], sem)
            return carry2

        lax.fori_loop(0, CHUNK, row_body, 0)
        pltpu.make_async_copy(w_hbm.at[pl.ds(0, CHUNK)],
                              out_hbm.at[pl.ds(off, CHUNK)], sem).wait()
        return carry

    lax.fori_loop(0, NCHUNK, chunk_body, 0)


def kernel(x, w):
    flat = x.reshape(-1).astype(jnp.int32)
    out = _emb_lookup(flat, w)
    return out.reshape(x.shape + (DIM,))
